# Initial kernel scaffold; baseline (speedup 1.0000x reference)
import jax
import jax.numpy as jnp
from jax.experimental import pallas as pl


def kernel(x, w1_p, b1_p, w2_p, b2_p, w3_p, b3_p, ws_p, bs_p):
    raise NotImplementedError("write your pallas kernel here")


# speedup vs baseline: 2.2220x; 2.2220x over previous
import functools

import jax
import jax.numpy as jnp
import numpy as np
from jax.experimental import pallas as pl
from jax.experimental.pallas import tpu as pltpu

CIN = 256
PL = 128
COUT = 512
H = W = 28
S = 2
HO = WO = 14
HPP = 15
PHROWS = 16 * HPP
NROWS = 4 * PHROWS
MROWS = HPP * HPP
MOUT = HO * WO


def _consts():
    m = np.zeros((4, 16, HPP), np.float32)
    for p in range(4):
        a, b = p // 2, p % 2
        for i in range(16):
            for j in range(HPP):
                if i < HPP and 1 <= 2 * i + a <= 28 and 1 <= 2 * j + b <= 28:
                    m[p, i, j] = 1.0
    mask = np.broadcast_to(m.reshape(NROWS, 1), (NROWS, PL)).copy()
    sc = np.zeros((MROWS, MOUT), np.float32)
    for i in range(HO):
        for j in range(WO):
            sc[HPP * i + j, i * WO + j] = 1.0
    return jnp.asarray(mask), jnp.asarray(sc)


def _body(x_ref, w1_ref, b1_ref, w2_ref, b2_ref, w3_ref, ws_ref, bsum_ref,
          mask_ref, sc_ref, o_ref, o1_scr):
    acc1 = jnp.dot(x_ref[0], w1_ref[...], preferred_element_type=jnp.float32)
    o1 = jnp.maximum(acc1 + b1_ref[...], 0.0) * mask_ref[...]
    o1_scr[...] = o1.astype(jnp.bfloat16)

    cols = []
    for kh in range(3):
        for kw in range(3):
            p = (kh % 2) * 2 + (kw % 2)
            base = p * PHROWS + (kh // 2) * HPP + (kw // 2)
            cols.append(o1_scr[pl.ds(base, MROWS), :])
    patches = jnp.concatenate(cols, axis=1)
    acc2 = jnp.dot(patches, w2_ref[...], preferred_element_type=jnp.float32)
    o2 = jnp.maximum(acc2 + b2_ref[...], 0.0).astype(jnp.bfloat16)

    xs = x_ref[0, pl.ds(3 * PHROWS, MROWS), :]
    acc3 = jnp.dot(o2, w3_ref[...], preferred_element_type=jnp.float32)
    accs = jnp.dot(xs, ws_ref[...], preferred_element_type=jnp.float32)
    o3 = jnp.maximum(acc3 + accs + bsum_ref[...], 0.0)

    o_ref[0] = jax.lax.dot_general(
        o3, sc_ref[...], (((0,), (0,)), ((), ())),
        preferred_element_type=jnp.float32)


def kernel(x, w1_p, b1_p, w2_p, b2_p, w3_p, b3_p, ws_p, bs_p):
    n = x.shape[0]
    mask, sc = _consts()

    xp = jnp.pad(x, ((0, 0), (0, 0), (1, 1), (1, 1)))
    xr = xp.reshape(n, CIN, HPP, S, HPP, S)
    xr = jnp.transpose(xr, (0, 3, 5, 2, 4, 1))
    xr = jnp.pad(xr, ((0, 0), (0, 0), (0, 0), (0, 1), (0, 0), (0, 0)))
    xrows = xr.reshape(n, NROWS, CIN).astype(jnp.bfloat16)

    w1 = w1_p.astype(jnp.bfloat16)
    w2 = w2_p.astype(jnp.bfloat16)
    w3 = w3_p.astype(jnp.bfloat16)
    ws = ws_p.astype(jnp.bfloat16)
    b1 = b1_p.reshape(1, PL)
    b2 = b2_p.reshape(1, PL)
    bsum = (b3_p + bs_p).reshape(1, COUT)

    flops = 2 * n * (NROWS * CIN * PL + MROWS * 9 * PL * PL
                     + MROWS * PL * COUT + MROWS * CIN * COUT
                     + MROWS * COUT * MOUT)
    bytes_accessed = (n * NROWS * CIN * 2 + n * COUT * MOUT * 4
                      + (w1.size + w2.size + w3.size + ws.size) * 2)
    const = lambda i: (0, 0)
    out = pl.pallas_call(
        _body,
        grid=(n,),
        in_specs=[
            pl.BlockSpec((1, NROWS, CIN), lambda i: (i, 0, 0)),
            pl.BlockSpec((CIN, PL), const),
            pl.BlockSpec((1, PL), const),
            pl.BlockSpec((9 * PL, PL), const),
            pl.BlockSpec((1, PL), const),
            pl.BlockSpec((PL, COUT), const),
            pl.BlockSpec((CIN, COUT), const),
            pl.BlockSpec((1, COUT), const),
            pl.BlockSpec((NROWS, PL), const),
            pl.BlockSpec((MROWS, MOUT), const),
        ],
        out_specs=pl.BlockSpec((1, COUT, MOUT), lambda i: (i, 0, 0)),
        out_shape=jax.ShapeDtypeStruct((n, COUT, MOUT), jnp.float32),
        scratch_shapes=[pltpu.VMEM((NROWS, PL), jnp.bfloat16)],
        compiler_params=pltpu.CompilerParams(
            dimension_semantics=("parallel",),
            vmem_limit_bytes=64 * 1024 * 1024),
        cost_estimate=pl.CostEstimate(flops=flops, transcendentals=0,
                                      bytes_accessed=bytes_accessed),
    )(xrows, w1, b1, w2, b2, w3, ws, bsum, mask, sc)
    return out.reshape(n, COUT, HO, WO)
